# baseline (speedup 1.0000x reference)
import functools

import jax
import jax.numpy as jnp
from jax import lax
from jax.experimental import pallas as pl
from jax.experimental.pallas import tpu as pltpu
from jax.experimental.pallas import tpu_sc as plsc

NUM_NODES = 50000
BATCH = 10000
NUM_EDGES = 320000
FEAT = 128

NC = 2
NS = 16
HF = FEAT // NC
K = 128
NCHUNK = 160
E_PAD = NS * NCHUNK * K
BATCHP = 10240
RPT = BATCHP // NS
CW = 16
ZROWS = 64
NBUF = 5
NGRP = NCHUNK // NBUF
SRC_BITS = 17
SRC_MASK = (1 << SRC_BITS) - 1


def _phase1_body(pk_hbm, tbl_hbm, out_hbm,
                 pk_v, idx_s, seg_s, rows_v, ones_v, zrow_v, zcnt_v,
                 acc_sh, cnt_sh, sem, sem2):
  c = lax.axis_index("c")
  s = lax.axis_index("s")

  zero16 = jnp.zeros((16,), jnp.float32)
  one16 = jnp.ones((16,), jnp.float32)

  def init_zrow(i, carry):
    for q in range(HF // 16):
      zrow_v[i, pl.ds(q * 16, 16)] = zero16
    return carry

  lax.fori_loop(0, ZROWS, init_zrow, 0)

  def init_zcnt(i, carry):
    zcnt_v[i, :] = zero16
    return carry

  lax.fori_loop(0, ZROWS, init_zcnt, 0)

  def init_ones(i, carry):
    ones_v[i, :] = one16
    return carry

  lax.fori_loop(0, K, init_ones, 0)

  row0 = s * RPT
  for r in range(RPT // ZROWS):
    pltpu.sync_copy(zrow_v, acc_sh.at[pl.ds(row0 + r * ZROWS, ZROWS)])
  for r in range(RPT // ZROWS):
    pltpu.sync_copy(zcnt_v, cnt_sh.at[pl.ds(row0 + r * ZROWS, ZROWS)])
  plsc.subcore_barrier()

  pltpu.sync_copy(pk_hbm.at[s], pk_v)

  def unpack(chunk, slot):
    for q in range(K // 16):
      col = pl.ds(q * 16, 16)
      p = pk_v[chunk, col]
      idx_s[slot, col] = (p & SRC_MASK) * 2 + c
      seg_s[slot, col] = lax.shift_right_logical(p, SRC_BITS)

  for b in range(NBUF):
    unpack(b, b)
    pltpu.async_copy(tbl_hbm.at[idx_s.at[b]], rows_v.at[b], sem.at[b])

  def wait_scatters(bq):
    pltpu.make_async_copy(rows_v.at[bq], acc_sh.at[seg_s.at[bq]],
                          sem2.at[bq]).wait()
    pltpu.make_async_copy(ones_v, cnt_sh.at[seg_s.at[bq]],
                          sem2.at[bq]).wait()

  def group(g, carry):
    for b in range(NBUF):
      j = g * NBUF + b
      pltpu.make_async_copy(tbl_hbm.at[idx_s.at[b]], rows_v.at[b],
                            sem.at[b]).wait()
      segrow = seg_s.at[b]
      pltpu.async_copy(rows_v.at[b], acc_sh.at[segrow], sem2.at[b],
                       add=True)
      pltpu.async_copy(ones_v, cnt_sh.at[segrow], sem2.at[b], add=True)

      bp = b - 1 if b > 0 else NBUF - 1
      if b == 0:
        @pl.when(g > 0)
        def _():
          wait_scatters(bp)
          unpack(j - 1 + NBUF, bp)
          pltpu.async_copy(tbl_hbm.at[idx_s.at[bp]], rows_v.at[bp],
                           sem.at[bp])
      else:
        wait_scatters(bp)

        @pl.when(g < NGRP - 1)
        def _():
          unpack(j - 1 + NBUF, bp)
          pltpu.async_copy(tbl_hbm.at[idx_s.at[bp]], rows_v.at[bp],
                           sem.at[bp])

    return carry

  lax.fori_loop(0, NGRP, group, 0)
  wait_scatters(NBUF - 1)
  plsc.subcore_barrier()

  def finish_block(b0, nrows):
    pltpu.sync_copy(acc_sh.at[pl.ds(b0, nrows)], zrow_v.at[pl.ds(0, nrows)])
    pltpu.sync_copy(cnt_sh.at[pl.ds(b0, nrows)], zcnt_v.at[pl.ds(0, nrows)])

    def row_fn(r, carry):
      cv = jnp.maximum(zcnt_v[r, :], 1.0)
      for q in range(HF // 16):
        col = pl.ds(q * 16, 16)
        zrow_v[r, col] = jnp.maximum(zrow_v[r, col] / cv, 0.0)
      return carry

    lax.fori_loop(0, nrows, row_fn, 0)
    pltpu.sync_copy(zrow_v.at[pl.ds(0, nrows)],
                    out_hbm.at[pl.ds(b0, nrows), pl.ds(c * HF, HF)])

  @pl.when(s < NS - 1)
  def _():
    for t in range(RPT // ZROWS):
      finish_block(row0 + t * ZROWS, ZROWS)

  @pl.when(s == NS - 1)
  def _():
    last0 = (NS - 1) * RPT
    nfull = (BATCH - last0) // ZROWS
    for t in range(nfull):
      finish_block(last0 + t * ZROWS, ZROWS)
    rem = BATCH - (last0 + nfull * ZROWS)
    finish_block(last0 + nfull * ZROWS, rem)


_phase1 = functools.partial(
    pl.kernel,
    out_type=jax.ShapeDtypeStruct((BATCH, FEAT), jnp.float32),
    mesh=plsc.VectorSubcoreMesh(
        core_axis_name="c", subcore_axis_name="s",
        num_cores=NC, num_subcores=NS),
    scratch_types=[
        pltpu.VMEM((NCHUNK, K), jnp.int32),
        pltpu.VMEM((NBUF, K), jnp.int32),
        pltpu.VMEM((NBUF, K), jnp.int32),
        pltpu.VMEM((NBUF, K, HF), jnp.float32),
        pltpu.VMEM((K, CW), jnp.float32),
        pltpu.VMEM((ZROWS, HF), jnp.float32),
        pltpu.VMEM((ZROWS, CW), jnp.float32),
        pltpu.VMEM_SHARED((BATCHP, HF), jnp.float32),
        pltpu.VMEM_SHARED((BATCHP, CW), jnp.float32),
        pltpu.SemaphoreType.DMA((NBUF,)),
        pltpu.SemaphoreType.DMA((NBUF,)),
    ],
    compiler_params=pltpu.CompilerParams(use_tc_tiling_on_sc=False),
)(_phase1_body)


@jax.jit
def kernel(neigh_src_ids, neigh_seg_ids, features_table):
  src = neigh_src_ids.astype(jnp.int32)
  seg = neigh_seg_ids.astype(jnp.int32)
  packed = jnp.bitwise_or(jnp.left_shift(seg, SRC_BITS), src)
  pad_word = jnp.int32((BATCHP - 1) << SRC_BITS)
  packed = jnp.concatenate(
      [packed, jnp.full((E_PAD - NUM_EDGES,), pad_word, jnp.int32)])
  pk3d = packed.reshape(NS, NCHUNK, K)
  tbl = features_table.reshape(NUM_NODES * NC, HF)
  return _phase1(pk3d, tbl)


# speedup vs baseline: 2.8127x; 2.8127x over previous
import functools

import jax
import jax.numpy as jnp
from jax import lax
from jax.experimental import pallas as pl
from jax.experimental.pallas import tpu as pltpu
from jax.experimental.pallas import tpu_sc as plsc

NUM_NODES = 50000
BATCH = 10000
NUM_EDGES = 320000
FEAT = 128

NC = 2
NS = 16
HF = FEAT // NC
E_TILE = NUM_EDGES // NS
K = 80
NCHUNK = E_TILE // K
BATCHP = 10240
RPT = BATCHP // NS
CW = 16
ZROWS = 64
NBUF = 10
NGRP = NCHUNK // NBUF
SRC_BITS = 17
SRC_MASK = (1 << SRC_BITS) - 1


def _phase1_body(pk_hbm, tbl_hbm, out_hbm,
                 pk_v, idx_s, seg_s, rows_v, ones_v, zrow_v, zcnt_v,
                 acc_sh, cnt_sh, sem, sem2):
  c = lax.axis_index("c")
  s = lax.axis_index("s")

  zero16 = jnp.zeros((16,), jnp.float32)
  one16 = jnp.ones((16,), jnp.float32)

  def init_zrow(i, carry):
    for q in range(HF // 16):
      zrow_v[i, pl.ds(q * 16, 16)] = zero16
    return carry

  lax.fori_loop(0, ZROWS, init_zrow, 0)

  def init_zcnt(i, carry):
    zcnt_v[i, :] = zero16
    return carry

  lax.fori_loop(0, ZROWS, init_zcnt, 0)

  def init_ones(i, carry):
    ones_v[i, :] = one16
    return carry

  lax.fori_loop(0, K, init_ones, 0)

  row0 = s * RPT
  for r in range(RPT // ZROWS):
    pltpu.sync_copy(zrow_v, acc_sh.at[pl.ds(row0 + r * ZROWS, ZROWS)])
  for r in range(RPT // ZROWS):
    pltpu.sync_copy(zcnt_v, cnt_sh.at[pl.ds(row0 + r * ZROWS, ZROWS)])
  plsc.subcore_barrier()

  pltpu.sync_copy(pk_hbm.at[s], pk_v)

  def unpack(chunk, slot):
    for q in range(K // 16):
      col = pl.ds(q * 16, 16)
      p = pk_v[chunk, col]
      idx_s[slot, col] = (p & SRC_MASK) * 2 + c
      seg_s[slot, col] = lax.shift_right_logical(p, SRC_BITS)

  for b in range(NBUF):
    unpack(b, b)
    pltpu.async_copy(tbl_hbm.at[idx_s.at[b]], rows_v.at[b], sem.at[b])

  def wait_scatters(bq):
    pltpu.make_async_copy(rows_v.at[bq], acc_sh.at[seg_s.at[bq]],
                          sem2.at[bq]).wait()
    pltpu.make_async_copy(ones_v, cnt_sh.at[seg_s.at[bq]],
                          sem2.at[bq]).wait()

  def group(g, carry):
    for b in range(NBUF):
      j = g * NBUF + b
      pltpu.make_async_copy(tbl_hbm.at[idx_s.at[b]], rows_v.at[b],
                            sem.at[b]).wait()
      segrow = seg_s.at[b]
      pltpu.async_copy(rows_v.at[b], acc_sh.at[segrow], sem2.at[b],
                       add=True)
      pltpu.async_copy(ones_v, cnt_sh.at[segrow], sem2.at[b], add=True)

      bp = b - 1 if b > 0 else NBUF - 1
      if b == 0:
        @pl.when(g > 0)
        def _():
          wait_scatters(bp)
          unpack(j - 1 + NBUF, bp)
          pltpu.async_copy(tbl_hbm.at[idx_s.at[bp]], rows_v.at[bp],
                           sem.at[bp])
      else:
        wait_scatters(bp)

        @pl.when(g < NGRP - 1)
        def _():
          unpack(j - 1 + NBUF, bp)
          pltpu.async_copy(tbl_hbm.at[idx_s.at[bp]], rows_v.at[bp],
                           sem.at[bp])

    return carry

  lax.fori_loop(0, NGRP, group, 0)
  wait_scatters(NBUF - 1)
  plsc.subcore_barrier()

  def finish_block(b0, nrows):
    pltpu.sync_copy(acc_sh.at[pl.ds(b0, nrows)], zrow_v.at[pl.ds(0, nrows)])
    pltpu.sync_copy(cnt_sh.at[pl.ds(b0, nrows)], zcnt_v.at[pl.ds(0, nrows)])

    def row_fn(r, carry):
      cv = jnp.maximum(zcnt_v[r, :], 1.0)
      for q in range(HF // 16):
        col = pl.ds(q * 16, 16)
        zrow_v[r, col] = jnp.maximum(zrow_v[r, col] / cv, 0.0)
      return carry

    lax.fori_loop(0, nrows, row_fn, 0)
    pltpu.sync_copy(zrow_v.at[pl.ds(0, nrows)],
                    out_hbm.at[pl.ds(b0, nrows), pl.ds(c * HF, HF)])

  @pl.when(s < NS - 1)
  def _():
    for t in range(RPT // ZROWS):
      finish_block(row0 + t * ZROWS, ZROWS)

  @pl.when(s == NS - 1)
  def _():
    last0 = (NS - 1) * RPT
    nfull = (BATCH - last0) // ZROWS
    for t in range(nfull):
      finish_block(last0 + t * ZROWS, ZROWS)
    rem = BATCH - (last0 + nfull * ZROWS)
    finish_block(last0 + nfull * ZROWS, rem)


_phase1 = functools.partial(
    pl.kernel,
    out_type=jax.ShapeDtypeStruct((BATCH, FEAT), jnp.float32),
    mesh=plsc.VectorSubcoreMesh(
        core_axis_name="c", subcore_axis_name="s",
        num_cores=NC, num_subcores=NS),
    scratch_types=[
        pltpu.VMEM((NCHUNK, K), jnp.int32),
        pltpu.VMEM((NBUF, K), jnp.int32),
        pltpu.VMEM((NBUF, K), jnp.int32),
        pltpu.VMEM((NBUF, K, HF), jnp.float32),
        pltpu.VMEM((K, CW), jnp.float32),
        pltpu.VMEM((ZROWS, HF), jnp.float32),
        pltpu.VMEM((ZROWS, CW), jnp.float32),
        pltpu.VMEM_SHARED((BATCHP, HF), jnp.float32),
        pltpu.VMEM_SHARED((BATCHP, CW), jnp.float32),
        pltpu.SemaphoreType.DMA((NBUF,)),
        pltpu.SemaphoreType.DMA((NBUF,)),
    ],
    compiler_params=pltpu.CompilerParams(use_tc_tiling_on_sc=False),
)(_phase1_body)


@jax.jit
def kernel(neigh_src_ids, neigh_seg_ids, features_table):
  src = neigh_src_ids.astype(jnp.int32)
  seg = neigh_seg_ids.astype(jnp.int32)
  packed = jnp.bitwise_or(jnp.left_shift(seg, SRC_BITS), src)
  pk3d = packed.reshape(NS, NCHUNK, K)
  tbl = features_table.reshape(NUM_NODES * NC, HF)
  return _phase1(pk3d, tbl)


# speedup vs baseline: 2.8727x; 1.0213x over previous
import functools

import jax
import jax.numpy as jnp
from jax import lax
from jax.experimental import pallas as pl
from jax.experimental.pallas import tpu as pltpu
from jax.experimental.pallas import tpu_sc as plsc

NUM_NODES = 50000
BATCH = 10000
NUM_EDGES = 320000
FEAT = 128

NC = 2
NS = 16
HF = FEAT // NC
E_TILE = NUM_EDGES // NS
K = 80
NCHUNK = E_TILE // K
BATCHP = 10240
RPT = BATCHP // NS
CW = 16
ZROWS = 64
NBUF = 5
NGRP = NCHUNK // NBUF
SRC_BITS = 17
SRC_MASK = (1 << SRC_BITS) - 1


def _phase1_body(pk_hbm, tbl_hbm, out_hbm,
                 pk_v, idx_s, seg_s, rows_v, ones_v, zrow_v, zcnt_v,
                 acc_sh, cnt_sh, sem, sem2, zsem):
  c = lax.axis_index("c")
  s = lax.axis_index("s")

  zero16 = jnp.zeros((16,), jnp.float32)
  one16 = jnp.ones((16,), jnp.float32)

  def init_zrow(i, carry):
    for q in range(HF // 16):
      zrow_v[i, pl.ds(q * 16, 16)] = zero16
    return carry

  lax.fori_loop(0, ZROWS, init_zrow, 0)

  def init_zcnt(i, carry):
    zcnt_v[i, :] = zero16
    return carry

  lax.fori_loop(0, ZROWS, init_zcnt, 0)

  def init_ones(i, carry):
    ones_v[i, :] = one16
    return carry

  lax.fori_loop(0, K, init_ones, 0)

  row0 = s * RPT

  def zero_copies():
    for r in range(RPT // ZROWS):
      yield pltpu.make_async_copy(
          zrow_v, acc_sh.at[pl.ds(row0 + r * ZROWS, ZROWS)], zsem)
    for r in range(RPT // ZROWS):
      yield pltpu.make_async_copy(
          zcnt_v, cnt_sh.at[pl.ds(row0 + r * ZROWS, ZROWS)], zsem)

  for cp in zero_copies():
    cp.start()

  pltpu.sync_copy(pk_hbm.at[s], pk_v)

  def unpack(chunk, slot):
    for q in range(K // 16):
      col = pl.ds(q * 16, 16)
      p = pk_v[chunk, col]
      idx_s[slot, col] = (p & SRC_MASK) * 2 + c
      seg_s[slot, col] = lax.shift_right_logical(p, SRC_BITS)

  for b in range(NBUF):
    unpack(b, b)
    pltpu.async_copy(tbl_hbm.at[idx_s.at[b]], rows_v.at[b], sem.at[b])

  for cp in zero_copies():
    cp.wait()
  plsc.subcore_barrier()

  def wait_scatters(bq):
    pltpu.make_async_copy(rows_v.at[bq], acc_sh.at[seg_s.at[bq]],
                          sem2.at[bq]).wait()
    pltpu.make_async_copy(ones_v, cnt_sh.at[seg_s.at[bq]],
                          sem2.at[bq]).wait()

  def group(g, carry):
    for b in range(NBUF):
      j = g * NBUF + b
      pltpu.make_async_copy(tbl_hbm.at[idx_s.at[b]], rows_v.at[b],
                            sem.at[b]).wait()
      segrow = seg_s.at[b]
      pltpu.async_copy(rows_v.at[b], acc_sh.at[segrow], sem2.at[b],
                       add=True)
      pltpu.async_copy(ones_v, cnt_sh.at[segrow], sem2.at[b], add=True)

      bp = b - 1 if b > 0 else NBUF - 1
      if b == 0:
        @pl.when(g > 0)
        def _():
          wait_scatters(bp)
          unpack(j - 1 + NBUF, bp)
          pltpu.async_copy(tbl_hbm.at[idx_s.at[bp]], rows_v.at[bp],
                           sem.at[bp])
      else:
        wait_scatters(bp)

        @pl.when(g < NGRP - 1)
        def _():
          unpack(j - 1 + NBUF, bp)
          pltpu.async_copy(tbl_hbm.at[idx_s.at[bp]], rows_v.at[bp],
                           sem.at[bp])

    return carry

  lax.fori_loop(0, NGRP, group, 0)
  wait_scatters(NBUF - 1)
  plsc.subcore_barrier()

  def finish_block(b0, nrows):
    pltpu.sync_copy(acc_sh.at[pl.ds(b0, nrows)], zrow_v.at[pl.ds(0, nrows)])
    pltpu.sync_copy(cnt_sh.at[pl.ds(b0, nrows)], zcnt_v.at[pl.ds(0, nrows)])

    def row_fn(r, carry):
      cv = jnp.maximum(zcnt_v[r, :], 1.0)
      for q in range(HF // 16):
        col = pl.ds(q * 16, 16)
        zrow_v[r, col] = jnp.maximum(zrow_v[r, col] / cv, 0.0)
      return carry

    lax.fori_loop(0, nrows, row_fn, 0)
    pltpu.sync_copy(zrow_v.at[pl.ds(0, nrows)],
                    out_hbm.at[pl.ds(b0, nrows), pl.ds(c * HF, HF)])

  @pl.when(s < NS - 1)
  def _():
    for t in range(RPT // ZROWS):
      finish_block(row0 + t * ZROWS, ZROWS)

  @pl.when(s == NS - 1)
  def _():
    last0 = (NS - 1) * RPT
    nfull = (BATCH - last0) // ZROWS
    for t in range(nfull):
      finish_block(last0 + t * ZROWS, ZROWS)
    rem = BATCH - (last0 + nfull * ZROWS)
    finish_block(last0 + nfull * ZROWS, rem)


_phase1 = functools.partial(
    pl.kernel,
    out_type=jax.ShapeDtypeStruct((BATCH, FEAT), jnp.float32),
    mesh=plsc.VectorSubcoreMesh(
        core_axis_name="c", subcore_axis_name="s",
        num_cores=NC, num_subcores=NS),
    scratch_types=[
        pltpu.VMEM((NCHUNK, K), jnp.int32),
        pltpu.VMEM((NBUF, K), jnp.int32),
        pltpu.VMEM((NBUF, K), jnp.int32),
        pltpu.VMEM((NBUF, K, HF), jnp.float32),
        pltpu.VMEM((K, CW), jnp.float32),
        pltpu.VMEM((ZROWS, HF), jnp.float32),
        pltpu.VMEM((ZROWS, CW), jnp.float32),
        pltpu.VMEM_SHARED((BATCHP, HF), jnp.float32),
        pltpu.VMEM_SHARED((BATCHP, CW), jnp.float32),
        pltpu.SemaphoreType.DMA((NBUF,)),
        pltpu.SemaphoreType.DMA((NBUF,)),
        pltpu.SemaphoreType.DMA,
    ],
    compiler_params=pltpu.CompilerParams(use_tc_tiling_on_sc=False),
)(_phase1_body)


@jax.jit
def kernel(neigh_src_ids, neigh_seg_ids, features_table):
  src = neigh_src_ids.astype(jnp.int32)
  seg = neigh_seg_ids.astype(jnp.int32)
  packed = jnp.bitwise_or(jnp.left_shift(seg, SRC_BITS), src)
  pk3d = packed.reshape(NS, NCHUNK, K)
  tbl = features_table.reshape(NUM_NODES * NC, HF)
  return _phase1(pk3d, tbl)


# speedup vs baseline: 2.9372x; 1.0225x over previous
import functools

import jax
import jax.numpy as jnp
from jax import lax
from jax.experimental import pallas as pl
from jax.experimental.pallas import tpu as pltpu
from jax.experimental.pallas import tpu_sc as plsc

NUM_NODES = 50000
BATCH = 10000
NUM_EDGES = 320000
FEAT = 128

NC = 2
NS = 16
HF = FEAT // NC
E_TILE = NUM_EDGES // NS
K = 80
NCHUNK = E_TILE // K
BATCHP = 10240
RPT = BATCHP // NS
CW = 16
ZROWS = 64
NBUF = 5
NGRP = NCHUNK // NBUF
SRC_BITS = 17
SRC_MASK = (1 << SRC_BITS) - 1


def _phase1_body(pk_hbm, tbl_hbm, out_hbm,
                 pk_v, idx_s, seg_s, rows_v, ones_v, zrow_v, zcnt_v,
                 zrow2_v, zcnt2_v, acc_sh, cnt_sh, sem, sem2, zsem,
                 esem, wsem):
  c = lax.axis_index("c")
  s = lax.axis_index("s")

  zero16 = jnp.zeros((16,), jnp.float32)
  one16 = jnp.ones((16,), jnp.float32)

  def init_zrow(i, carry):
    for q in range(HF // 16):
      zrow_v[i, pl.ds(q * 16, 16)] = zero16
    return carry

  lax.fori_loop(0, ZROWS, init_zrow, 0)

  def init_zcnt(i, carry):
    zcnt_v[i, :] = zero16
    return carry

  lax.fori_loop(0, ZROWS, init_zcnt, 0)

  def init_ones(i, carry):
    ones_v[i, :] = one16
    return carry

  lax.fori_loop(0, K, init_ones, 0)

  row0 = s * RPT

  def zero_copies():
    for r in range(RPT // ZROWS):
      yield pltpu.make_async_copy(
          zrow_v, acc_sh.at[pl.ds(row0 + r * ZROWS, ZROWS)], zsem)
    for r in range(RPT // ZROWS):
      yield pltpu.make_async_copy(
          zcnt_v, cnt_sh.at[pl.ds(row0 + r * ZROWS, ZROWS)], zsem)

  for cp in zero_copies():
    cp.start()

  pltpu.sync_copy(pk_hbm.at[s], pk_v)

  def unpack(chunk, slot):
    for q in range(K // 16):
      col = pl.ds(q * 16, 16)
      p = pk_v[chunk, col]
      idx_s[slot, col] = (p & SRC_MASK) * 2 + c
      seg_s[slot, col] = lax.shift_right_logical(p, SRC_BITS)

  for b in range(NBUF):
    unpack(b, b)
    pltpu.async_copy(tbl_hbm.at[idx_s.at[b]], rows_v.at[b], sem.at[b])

  for cp in zero_copies():
    cp.wait()
  plsc.subcore_barrier()

  def wait_scatters(bq):
    pltpu.make_async_copy(rows_v.at[bq], acc_sh.at[seg_s.at[bq]],
                          sem2.at[bq]).wait()
    pltpu.make_async_copy(ones_v, cnt_sh.at[seg_s.at[bq]],
                          sem2.at[bq]).wait()

  def group(g, carry):
    for b in range(NBUF):
      j = g * NBUF + b
      pltpu.make_async_copy(tbl_hbm.at[idx_s.at[b]], rows_v.at[b],
                            sem.at[b]).wait()
      segrow = seg_s.at[b]
      pltpu.async_copy(rows_v.at[b], acc_sh.at[segrow], sem2.at[b],
                       add=True)
      pltpu.async_copy(ones_v, cnt_sh.at[segrow], sem2.at[b], add=True)

      bp = b - 1 if b > 0 else NBUF - 1
      if b == 0:
        @pl.when(g > 0)
        def _():
          wait_scatters(bp)
          unpack(j - 1 + NBUF, bp)
          pltpu.async_copy(tbl_hbm.at[idx_s.at[bp]], rows_v.at[bp],
                           sem.at[bp])
      else:
        wait_scatters(bp)

        @pl.when(g < NGRP - 1)
        def _():
          unpack(j - 1 + NBUF, bp)
          pltpu.async_copy(tbl_hbm.at[idx_s.at[bp]], rows_v.at[bp],
                           sem.at[bp])

    return carry

  lax.fori_loop(0, NGRP, group, 0)
  wait_scatters(NBUF - 1)
  plsc.subcore_barrier()

  def run_blocks(blocks):
    n = len(blocks)
    zr = [zrow_v, zrow2_v]
    zc = [zcnt_v, zcnt2_v]

    def read_copies(t):
      b0, nr = blocks[t]
      p = t % 2
      return (pltpu.make_async_copy(acc_sh.at[pl.ds(b0, nr)],
                                    zr[p].at[pl.ds(0, nr)], esem.at[p]),
              pltpu.make_async_copy(cnt_sh.at[pl.ds(b0, nr)],
                                    zc[p].at[pl.ds(0, nr)], esem.at[p]))

    def wb_copy(t):
      b0, nr = blocks[t]
      p = t % 2
      return pltpu.make_async_copy(
          zr[p].at[pl.ds(0, nr)],
          out_hbm.at[pl.ds(b0, nr), pl.ds(c * HF, HF)], wsem.at[p])

    for cp in read_copies(0):
      cp.start()
    for t in range(n):
      p = t % 2
      b0, nr = blocks[t]
      if t + 1 < n:
        if t >= 1:
          wb_copy(t - 1).wait()
        for cp in read_copies(t + 1):
          cp.start()
      for cp in read_copies(t):
        cp.wait()

      def row_fn(r, carry):
        cv = jnp.maximum(zc[p][r, :], 1.0)
        for q in range(HF // 16):
          col = pl.ds(q * 16, 16)
          zr[p][r, col] = jnp.maximum(zr[p][r, col] / cv, 0.0)
        return carry

      lax.fori_loop(0, nr, row_fn, 0)
      wb_copy(t).start()
    if n >= 2:
      wb_copy(n - 2).wait()
    wb_copy(n - 1).wait()

  @pl.when(s < NS - 1)
  def _():
    run_blocks([(row0 + t * ZROWS, ZROWS) for t in range(RPT // ZROWS)])

  @pl.when(s == NS - 1)
  def _():
    last0 = (NS - 1) * RPT
    nfull = (BATCH - last0) // ZROWS
    rem = BATCH - (last0 + nfull * ZROWS)
    run_blocks([(last0 + t * ZROWS, ZROWS) for t in range(nfull)]
               + [(last0 + nfull * ZROWS, rem)])


_phase1 = functools.partial(
    pl.kernel,
    out_type=jax.ShapeDtypeStruct((BATCH, FEAT), jnp.float32),
    mesh=plsc.VectorSubcoreMesh(
        core_axis_name="c", subcore_axis_name="s",
        num_cores=NC, num_subcores=NS),
    scratch_types=[
        pltpu.VMEM((NCHUNK, K), jnp.int32),
        pltpu.VMEM((NBUF, K), jnp.int32),
        pltpu.VMEM((NBUF, K), jnp.int32),
        pltpu.VMEM((NBUF, K, HF), jnp.float32),
        pltpu.VMEM((K, CW), jnp.float32),
        pltpu.VMEM((ZROWS, HF), jnp.float32),
        pltpu.VMEM((ZROWS, CW), jnp.float32),
        pltpu.VMEM((ZROWS, HF), jnp.float32),
        pltpu.VMEM((ZROWS, CW), jnp.float32),
        pltpu.VMEM_SHARED((BATCHP, HF), jnp.float32),
        pltpu.VMEM_SHARED((BATCHP, CW), jnp.float32),
        pltpu.SemaphoreType.DMA((NBUF,)),
        pltpu.SemaphoreType.DMA((NBUF,)),
        pltpu.SemaphoreType.DMA,
        pltpu.SemaphoreType.DMA((2,)),
        pltpu.SemaphoreType.DMA((2,)),
    ],
    compiler_params=pltpu.CompilerParams(use_tc_tiling_on_sc=False),
)(_phase1_body)


@jax.jit
def kernel(neigh_src_ids, neigh_seg_ids, features_table):
  src = neigh_src_ids.astype(jnp.int32)
  seg = neigh_seg_ids.astype(jnp.int32)
  packed = jnp.bitwise_or(jnp.left_shift(seg, SRC_BITS), src)
  pk3d = packed.reshape(NS, NCHUNK, K)
  tbl = features_table.reshape(NUM_NODES * NC, HF)
  return _phase1(pk3d, tbl)
